# Initial kernel scaffold; baseline (speedup 1.0000x reference)
import functools

import jax
import jax.numpy as jnp
from jax import lax
from jax.experimental import pallas as pl
from jax.experimental.pallas import tpu as pltpu
from jax.experimental.pallas import tpu_sc as plsc

B = 4096
L = 200
D = 32
N = B * L
NUM_WORKERS = 32
ROWS_PER_WORKER = N // NUM_WORKERS
GATHER_ROWS = 128
K = 20
CHUNK = GATHER_ROWS * K
NUM_CHUNKS = ROWS_PER_WORKER // CHUNK


def _sc_body(idx_hbm, table_hbm, out_hbm, idx_v, rows_v, sem):
    wid = lax.axis_index("s") * 2 + lax.axis_index("c")
    base = wid * ROWS_PER_WORKER

    def chunk_body(c, carry):
        row0 = base + c * CHUNK
        pltpu.sync_copy(idx_hbm.at[pl.ds(row0 // GATHER_ROWS, K)], idx_v)
        copies = []
        for j in range(K):
            copies.append(
                pltpu.async_copy(
                    table_hbm.at[idx_v.at[j]],
                    rows_v.at[pl.ds(j * GATHER_ROWS, GATHER_ROWS)],
                    sem,
                )
            )
        for cp in copies:
            cp.wait()
        pltpu.sync_copy(rows_v, out_hbm.at[pl.ds(row0, CHUNK)])
        return carry

    lax.fori_loop(0, NUM_CHUNKS, chunk_body, 0)

    @pl.when(wid == 0)
    def _():
        zero = jnp.zeros((16,), jnp.float32)

        def zrow(i, carry):
            rows_v[i, pl.ds(0, 16)] = zero
            rows_v[i, pl.ds(16, 16)] = zero
            return carry

        lax.fori_loop(0, L, zrow, 0)
        pltpu.sync_copy(rows_v.at[pl.ds(0, L)], out_hbm.at[pl.ds(0, L)])


@jax.jit
def _sc_gather(idx_rows, table):
    mesh = plsc.VectorSubcoreMesh(core_axis_name="c", subcore_axis_name="s")
    run = functools.partial(
        pl.kernel,
        mesh=mesh,
        out_type=jax.ShapeDtypeStruct((N, D), jnp.float32),
        scratch_types=[
            pltpu.VMEM((K, GATHER_ROWS), jnp.int32),
            pltpu.VMEM((CHUNK, D), jnp.float32),
            pltpu.SemaphoreType.DMA,
        ],
    )(_sc_body)
    return run(idx_rows, table)


def kernel(y, table):
    yf = y.reshape(-1).astype(jnp.int32)
    idx = jnp.concatenate([jnp.zeros((L,), jnp.int32), yf[:-L]])
    idx_rows = idx.reshape(N // GATHER_ROWS, GATHER_ROWS)
    out = _sc_gather(idx_rows, table)
    return out.reshape(B, L, D)


# speedup vs baseline: 1.5525x; 1.5525x over previous
import functools

import jax
import jax.numpy as jnp
from jax import lax
from jax.experimental import pallas as pl
from jax.experimental.pallas import tpu as pltpu
from jax.experimental.pallas import tpu_sc as plsc

B = 4096
L = 200
D = 32
N = B * L
NUM_WORKERS = 32
ROWS_PER_WORKER = N // NUM_WORKERS
GATHER_ROWS = 128
K = 20
CHUNK = GATHER_ROWS * K
NUM_CHUNKS = ROWS_PER_WORKER // CHUNK


def _sc_body(idx_hbm, table_hbm, out_hbm, idx_v, rows_v, sem):
    wid = lax.axis_index("s") * 2 + lax.axis_index("c")
    base = pl.multiple_of(wid * ROWS_PER_WORKER, CHUNK)
    idx_row0 = pl.multiple_of(base // GATHER_ROWS, 8)

    pltpu.sync_copy(idx_hbm.at[pl.ds(idx_row0, ROWS_PER_WORKER // GATHER_ROWS)],
                    idx_v)

    def chunk_body(c, carry):
        row0 = pl.multiple_of(base + c * CHUNK, CHUNK)
        copies = []
        for j in range(K):
            copies.append(
                pltpu.async_copy(
                    table_hbm.at[idx_v.at[c * K + j]],
                    rows_v.at[pl.ds(j * GATHER_ROWS, GATHER_ROWS)],
                    sem,
                )
            )
        for cp in copies:
            cp.wait()
        pltpu.sync_copy(rows_v, out_hbm.at[pl.ds(row0, CHUNK)])
        return carry

    lax.fori_loop(0, NUM_CHUNKS, chunk_body, 0)

    @pl.when(wid == 0)
    def _():
        zero = jnp.zeros((16,), jnp.float32)

        def zrow(i, carry):
            rows_v[i, pl.ds(0, 16)] = zero
            rows_v[i, pl.ds(16, 16)] = zero
            return carry

        lax.fori_loop(0, L, zrow, 0)
        pltpu.sync_copy(rows_v.at[pl.ds(0, L)], out_hbm.at[pl.ds(0, L)])


@jax.jit
def _sc_gather(idx_rows, table):
    mesh = plsc.VectorSubcoreMesh(core_axis_name="c", subcore_axis_name="s")
    run = functools.partial(
        pl.kernel,
        mesh=mesh,
        out_type=jax.ShapeDtypeStruct((N, D), jnp.float32),
        scratch_types=[
            pltpu.VMEM((ROWS_PER_WORKER // GATHER_ROWS, GATHER_ROWS), jnp.int32),
            pltpu.VMEM((CHUNK, D), jnp.float32),
            pltpu.SemaphoreType.DMA,
        ],
        compiler_params=pltpu.CompilerParams(use_tc_tiling_on_sc=False),
    )(_sc_body)
    return run(idx_rows, table)


def kernel(y, table):
    yf = y.reshape(-1).astype(jnp.int32)
    idx = jnp.concatenate([jnp.zeros((L,), jnp.int32), yf[:-L]])
    idx_rows = idx.reshape(N // GATHER_ROWS, GATHER_ROWS)
    out = _sc_gather(idx_rows, table)
    return out.reshape(B, L, D)


# speedup vs baseline: 1.5614x; 1.0057x over previous
import functools

import jax
import jax.numpy as jnp
from jax import lax
from jax.experimental import pallas as pl
from jax.experimental.pallas import tpu as pltpu
from jax.experimental.pallas import tpu_sc as plsc

B = 4096
L = 200
D = 32
N = B * L
NUM_WORKERS = 32
ROWS_PER_WORKER = N // NUM_WORKERS
GATHER_ROWS = 128
K = 10
CHUNK = GATHER_ROWS * K
NUM_CHUNKS = ROWS_PER_WORKER // CHUNK
NUM_STEPS = NUM_CHUNKS // 2


def _sc_body(idx_hbm, table_hbm, out_hbm, idx_v, rows0, rows1, gsem0, gsem1,
             ssem0, ssem1):
    wid = lax.axis_index("s") * 2 + lax.axis_index("c")
    base = pl.multiple_of(wid * ROWS_PER_WORKER, CHUNK)
    idx_row0 = pl.multiple_of(base // GATHER_ROWS, 8)

    pltpu.sync_copy(idx_hbm.at[pl.ds(idx_row0, ROWS_PER_WORKER // GATHER_ROWS)],
                    idx_v)

    def fire(c, buf, gsem):
        for j in range(K):
            pltpu.async_copy(
                table_hbm.at[idx_v.at[c * K + j]],
                buf.at[pl.ds(j * GATHER_ROWS, GATHER_ROWS)],
                gsem,
            )

    def drain_gathers(buf, gsem):
        pltpu.make_async_copy(table_hbm.at[pl.ds(0, CHUNK)], buf, gsem).wait()

    def store(c, buf, ssem):
        row0 = pl.multiple_of(base + c * CHUNK, CHUNK)
        pltpu.async_copy(buf, out_hbm.at[pl.ds(row0, CHUNK)], ssem)

    def wait_store(buf, ssem):
        pltpu.make_async_copy(buf, out_hbm.at[pl.ds(base, CHUNK)], ssem).wait()

    fire(0, rows0, gsem0)

    def step(i, carry):
        c0 = pl.multiple_of(2 * i, 2)

        @pl.when(i > 0)
        def _():
            wait_store(rows1, ssem1)
        fire(c0 + 1, rows1, gsem1)
        drain_gathers(rows0, gsem0)
        store(c0, rows0, ssem0)

        @pl.when(i < NUM_STEPS - 1)
        def _():
            wait_store(rows0, ssem0)
            fire(c0 + 2, rows0, gsem0)
        drain_gathers(rows1, gsem1)
        store(c0 + 1, rows1, ssem1)
        return carry

    lax.fori_loop(0, NUM_STEPS, step, 0)
    wait_store(rows0, ssem0)
    wait_store(rows1, ssem1)

    @pl.when(wid == 0)
    def _():
        zero = jnp.zeros((16,), jnp.float32)

        def zrow(i, carry):
            rows0[i, pl.ds(0, 16)] = zero
            rows0[i, pl.ds(16, 16)] = zero
            return carry

        lax.fori_loop(0, L, zrow, 0)
        pltpu.sync_copy(rows0.at[pl.ds(0, L)], out_hbm.at[pl.ds(0, L)])


@jax.jit
def _sc_gather(idx_rows, table):
    mesh = plsc.VectorSubcoreMesh(core_axis_name="c", subcore_axis_name="s")
    run = functools.partial(
        pl.kernel,
        mesh=mesh,
        out_type=jax.ShapeDtypeStruct((N, D), jnp.float32),
        scratch_types=[
            pltpu.VMEM((ROWS_PER_WORKER // GATHER_ROWS, GATHER_ROWS), jnp.int32),
            pltpu.VMEM((CHUNK, D), jnp.float32),
            pltpu.VMEM((CHUNK, D), jnp.float32),
            pltpu.SemaphoreType.DMA,
            pltpu.SemaphoreType.DMA,
            pltpu.SemaphoreType.DMA,
            pltpu.SemaphoreType.DMA,
        ],
        compiler_params=pltpu.CompilerParams(use_tc_tiling_on_sc=False),
    )(_sc_body)
    return run(idx_rows, table)


def kernel(y, table):
    yf = y.reshape(-1).astype(jnp.int32)
    idx = jnp.concatenate([jnp.zeros((L,), jnp.int32), yf[:-L]])
    idx_rows = idx.reshape(N // GATHER_ROWS, GATHER_ROWS)
    out = _sc_gather(idx_rows, table)
    return out.reshape(B, L, D)
